# baseline (speedup 1.0000x reference)
import functools

import jax
import jax.numpy as jnp
from jax import lax
from jax.experimental import pallas as pl
from jax.experimental.pallas import tpu as pltpu
from jax.experimental.pallas import tpu_sc as plsc

B, T, D = 64, 200, 4096
MAX_FRAMES = 1000
N = B * T
NC, NS, L = 2, 16, 16
NW = NC * NS
ROWS_PER_W = N // NW
R = 8
CHUNKS = ROWS_PER_W // R

_mesh = plsc.VectorSubcoreMesh(core_axis_name="c", subcore_axis_name="s")


@functools.partial(
    pl.kernel,
    out_type=jax.ShapeDtypeStruct((N, D), jnp.float32),
    mesh=_mesh,
    scratch_types=[
        pltpu.VMEM((ROWS_PER_W,), jnp.int32),
        pltpu.VMEM((R, D), jnp.float32),
        pltpu.VMEM((R, D), jnp.float32),
        pltpu.SemaphoreType.DMA,
        pltpu.SemaphoreType.DMA,
    ],
)
def _pe_add(x_hbm, idx_hbm, pe_hbm, out_hbm, idx_v, xbuf, pebuf, sem_x, sem_pe):
    wid = lax.axis_index("s") * NC + lax.axis_index("c")
    base = wid * ROWS_PER_W

    pltpu.sync_copy(idx_hbm.at[pl.ds(base, ROWS_PER_W)], idx_v)

    @plsc.parallel_loop(0, ROWS_PER_W, step=L, unroll=4)
    def _clamp(i):
        v = idx_v[pl.ds(i, L)]
        idx_v[pl.ds(i, L)] = jnp.minimum(jnp.maximum(v, 0), MAX_FRAMES - 1)

    def chunk_body(c, carry):
        row0 = base + c * R
        cp_x = pltpu.async_copy(x_hbm.at[pl.ds(row0, R)], xbuf, sem_x)
        cp_pe = pltpu.async_copy(pe_hbm.at[idx_v.at[pl.ds(c * R, R)]], pebuf, sem_pe)
        cp_x.wait()
        cp_pe.wait()
        for r in range(R):
            @plsc.parallel_loop(0, D, step=L, unroll=8)
            def _acc(j):
                plsc.addupdate(xbuf.at[r, pl.ds(j, L)], pebuf[r, pl.ds(j, L)])
        pltpu.sync_copy(xbuf, out_hbm.at[pl.ds(row0, R)])
        return carry

    lax.fori_loop(0, CHUNKS, chunk_body, 0)


def kernel(x, frame_indices, pe):
    xf = x.reshape(N, D)
    idx = frame_indices.reshape(N).astype(jnp.int32)
    out = _pe_add(xf, idx, pe)
    return out.reshape(B, T, D)


# speedup vs baseline: 1.3815x; 1.3815x over previous
import functools

import jax
import jax.numpy as jnp
from jax import lax
from jax.experimental import pallas as pl
from jax.experimental.pallas import tpu as pltpu
from jax.experimental.pallas import tpu_sc as plsc

B, T, D = 64, 200, 4096
MAX_FRAMES = 1000
N = B * T
NC, NS, L = 2, 16, 16
NW = NC * NS
ROWS_PER_W = N // NW
R = 8
CHUNKS = ROWS_PER_W // R

_mesh = plsc.VectorSubcoreMesh(core_axis_name="c", subcore_axis_name="s")


@functools.partial(
    pl.kernel,
    out_type=jax.ShapeDtypeStruct((N, D), jnp.float32),
    mesh=_mesh,
    scratch_types=[
        pltpu.VMEM((ROWS_PER_W,), jnp.int32),
        pltpu.VMEM((R, D), jnp.float32),
        pltpu.VMEM((R, D), jnp.float32),
        pltpu.VMEM((R, D), jnp.float32),
        pltpu.SemaphoreType.DMA,
        pltpu.SemaphoreType.DMA,
        pltpu.SemaphoreType.DMA,
        pltpu.SemaphoreType.DMA,
        pltpu.SemaphoreType.DMA,
    ],
)
def _pe_add(x_hbm, idx_hbm, pe_hbm, out_hbm, idx_v,
            xb0, xb1, pb, sx0, sx1, sg, so0, so1):
    wid = lax.axis_index("s") * NC + lax.axis_index("c")
    base = wid * ROWS_PER_W
    xb = (xb0, xb1)
    sx = (sx0, sx1)
    so = (so0, so1)

    pltpu.sync_copy(idx_hbm.at[pl.ds(base, ROWS_PER_W)], idx_v)

    @plsc.parallel_loop(0, ROWS_PER_W, step=L, unroll=4)
    def _clamp(i):
        v = idx_v[pl.ds(i, L)]
        idx_v[pl.ds(i, L)] = jnp.minimum(jnp.maximum(v, 0), MAX_FRAMES - 1)

    def start_x(c, b):
        pltpu.async_copy(x_hbm.at[pl.ds(base + c * R, R)], xb[b], sx[b])

    def start_g(c):
        pltpu.async_copy(pe_hbm.at[idx_v.at[pl.ds(c * R, R)]], pb, sg)

    start_x(0, 0)
    start_g(0)

    @pl.loop(0, CHUNKS, step=2)
    def _chunks(cc):
        for b in (0, 1):
            c = cc + b
            row0 = base + c * R

            @pl.when(c >= 1)
            def _():
                pltpu.make_async_copy(
                    xb[1 - b], out_hbm.at[pl.ds(row0 - R, R)], so[1 - b]).wait()

            @pl.when(c + 1 < CHUNKS)
            def _():
                start_x(c + 1, 1 - b)

            pltpu.make_async_copy(x_hbm.at[pl.ds(row0, R)], xb[b], sx[b]).wait()
            pltpu.make_async_copy(
                pe_hbm.at[idx_v.at[pl.ds(c * R, R)]], pb, sg).wait()

            for r in range(R):
                @plsc.parallel_loop(0, D, step=L, unroll=8)
                def _acc(j):
                    plsc.addupdate(xb[b].at[r, pl.ds(j, L)], pb[r, pl.ds(j, L)])

            pltpu.async_copy(xb[b], out_hbm.at[pl.ds(row0, R)], so[b])

            @pl.when(c + 1 < CHUNKS)
            def _():
                start_g(c + 1)

    fb = (CHUNKS - 1) & 1
    pltpu.make_async_copy(
        xb[fb], out_hbm.at[pl.ds(base + (CHUNKS - 1) * R, R)], so[fb]).wait()


def kernel(x, frame_indices, pe):
    xf = x.reshape(N, D)
    idx = frame_indices.reshape(N).astype(jnp.int32)
    out = _pe_add(xf, idx, pe)
    return out.reshape(B, T, D)


# speedup vs baseline: 1.4791x; 1.0706x over previous
import functools

import jax
import jax.numpy as jnp
from jax import lax
from jax.experimental import pallas as pl
from jax.experimental.pallas import tpu as pltpu
from jax.experimental.pallas import tpu_sc as plsc

B, T, D = 64, 200, 4096
MAX_FRAMES = 1000
N = B * T
NC, NS, L = 2, 16, 16
NW = NC * NS
ROWS_PER_W = N // NW
R = 8
CHUNKS = ROWS_PER_W // R

_mesh = plsc.VectorSubcoreMesh(core_axis_name="c", subcore_axis_name="s")


@functools.partial(
    pl.kernel,
    out_type=jax.ShapeDtypeStruct((N, D), jnp.float32),
    mesh=_mesh,
    scratch_types=[
        pltpu.VMEM((ROWS_PER_W,), jnp.int32),
        pltpu.VMEM((R, D), jnp.float32),
        pltpu.VMEM((R, D), jnp.float32),
        pltpu.VMEM((R, D // 2), jnp.int32),
        pltpu.VMEM((R, D // 2), jnp.int32),
        pltpu.SemaphoreType.DMA,
        pltpu.SemaphoreType.DMA,
        pltpu.SemaphoreType.DMA,
        pltpu.SemaphoreType.DMA,
        pltpu.SemaphoreType.DMA,
        pltpu.SemaphoreType.DMA,
    ],
)
def _pe_add(x_hbm, idx_hbm, pe_hbm, out_hbm, idx_v,
            xb0, xb1, pb0, pb1, sx0, sx1, sg0, sg1, so0, so1):
    wid = lax.axis_index("s") * NC + lax.axis_index("c")
    base = wid * ROWS_PER_W
    xb = (xb0, xb1)
    pb = (pb0, pb1)
    sx = (sx0, sx1)
    sg = (sg0, sg1)
    so = (so0, so1)

    pltpu.sync_copy(idx_hbm.at[pl.ds(base, ROWS_PER_W)], idx_v)

    @plsc.parallel_loop(0, ROWS_PER_W, step=L, unroll=4)
    def _clamp(i):
        v = idx_v[pl.ds(i, L)]
        idx_v[pl.ds(i, L)] = jnp.minimum(jnp.maximum(v, 0), MAX_FRAMES - 1)

    def start_in(c, b):
        pltpu.async_copy(x_hbm.at[pl.ds(base + c * R, R)], xb[b], sx[b])
        pltpu.async_copy(pe_hbm.at[idx_v.at[pl.ds(c * R, R)]], pb[b], sg[b])

    start_in(0, 0)

    @pl.loop(0, CHUNKS, step=2)
    def _chunks(cc):
        for b in (0, 1):
            c = cc + b
            row0 = base + c * R

            @pl.when(c >= 1)
            def _():
                pltpu.make_async_copy(
                    xb[1 - b], out_hbm.at[pl.ds(row0 - R, R)], so[1 - b]).wait()

            @pl.when(c + 1 < CHUNKS)
            def _():
                start_in(c + 1, 1 - b)

            pltpu.make_async_copy(x_hbm.at[pl.ds(row0, R)], xb[b], sx[b]).wait()
            pltpu.make_async_copy(
                pe_hbm.at[idx_v.at[pl.ds(c * R, R)]], pb[b], sg[b]).wait()

            for r in range(R):
                @plsc.parallel_loop(0, D // 2, step=L, unroll=8)
                def _acc(k):
                    u = pb[b][r, pl.ds(k, L)]
                    lo = lax.bitcast_convert_type(u << 16, jnp.float32)
                    hi = lax.bitcast_convert_type(u & jnp.int32(-65536), jnp.float32)
                    plsc.addupdate(xb[b].at[r, pl.ds(2 * k, L)], lo)
                    plsc.addupdate(xb[b].at[r, pl.ds(2 * k + L, L)], hi)

            pltpu.async_copy(xb[b], out_hbm.at[pl.ds(row0, R)], so[b])

    fb = (CHUNKS - 1) & 1
    pltpu.make_async_copy(
        xb[fb], out_hbm.at[pl.ds(base + (CHUNKS - 1) * R, R)], so[fb]).wait()


def kernel(x, frame_indices, pe):
    xf = x.reshape(N, D)
    idx = frame_indices.reshape(N).astype(jnp.int32)
    pe_bf = (pe.astype(jnp.bfloat16)
             .reshape(MAX_FRAMES, D // 32, 2, 16)
             .transpose(0, 1, 3, 2)
             .reshape(MAX_FRAMES, D // 2, 2))
    pe_pk = jax.lax.bitcast_convert_type(pe_bf, jnp.int32)
    out = _pe_add(xf, idx, pe_pk)
    return out.reshape(B, T, D)


# speedup vs baseline: 1.5039x; 1.0168x over previous
import functools

import jax
import jax.numpy as jnp
from jax import lax
from jax.experimental import pallas as pl
from jax.experimental.pallas import tpu as pltpu
from jax.experimental.pallas import tpu_sc as plsc

B, T, D = 64, 200, 4096
MAX_FRAMES = 1000
N = B * T
NC, NS, L = 2, 16, 16
NW = NC * NS
ROWS_PER_W = N // NW
R = 4
CHUNKS = ROWS_PER_W // R
NB = 4

_mesh = plsc.VectorSubcoreMesh(core_axis_name="c", subcore_axis_name="s")


@functools.partial(
    pl.kernel,
    out_type=jax.ShapeDtypeStruct((N, D), jnp.float32),
    mesh=_mesh,
    scratch_types=[
        pltpu.VMEM((CHUNKS * 8,), jnp.int32),
        tuple(pltpu.VMEM((R, D), jnp.float32) for _ in range(NB)),
        tuple(pltpu.VMEM((R, D // 2), jnp.int32) for _ in range(NB)),
        tuple(pltpu.SemaphoreType.DMA for _ in range(3 * NB)),
    ],
)
def _pe_add(x_hbm, idx_hbm, pe_hbm, out_hbm, idx_v, xb, pb, sems):
    wid = lax.axis_index("s") * NC + lax.axis_index("c")
    base = wid * ROWS_PER_W
    sx = sems[0:NB]
    sg = sems[NB:2 * NB]
    so = sems[2 * NB:3 * NB]

    pltpu.sync_copy(idx_hbm.at[pl.ds(wid * CHUNKS * 8, CHUNKS * 8)], idx_v)

    @plsc.parallel_loop(0, CHUNKS * 8, step=L, unroll=4)
    def _clamp(i):
        v = idx_v[pl.ds(i, L)]
        idx_v[pl.ds(i, L)] = jnp.minimum(jnp.maximum(v, 0), MAX_FRAMES - 1)

    def start_in(c, b):
        pltpu.async_copy(x_hbm.at[pl.ds(base + c * R, R)], xb[b], sx[b])
        pltpu.async_copy(pe_hbm.at[idx_v.at[pl.ds(c * 8, R)]], pb[b], sg[b])

    start_in(0, 0)
    start_in(1, 1)

    @pl.loop(0, CHUNKS, step=NB)
    def _chunks(cc):
        for b in range(NB):
            c = cc + b
            row0 = base + c * R

            @pl.when(c >= 2)
            def _():
                pltpu.make_async_copy(
                    xb[(b + 2) % NB],
                    out_hbm.at[pl.ds(row0 - 2 * R, R)], so[(b + 2) % NB]).wait()

            @pl.when(c + 2 < CHUNKS)
            def _():
                start_in(c + 2, (b + 2) % NB)

            pltpu.make_async_copy(x_hbm.at[pl.ds(row0, R)], xb[b], sx[b]).wait()
            pltpu.make_async_copy(
                pe_hbm.at[idx_v.at[pl.ds(c * 8, R)]], pb[b], sg[b]).wait()

            for r in range(R):
                @plsc.parallel_loop(0, D // 2, step=L, unroll=8)
                def _acc(k):
                    u = pb[b][r, pl.ds(k, L)]
                    lo = lax.bitcast_convert_type(u << 16, jnp.float32)
                    hi = lax.bitcast_convert_type(u & jnp.int32(-65536), jnp.float32)
                    plsc.addupdate(xb[b].at[r, pl.ds(2 * k, L)], lo)
                    plsc.addupdate(xb[b].at[r, pl.ds(2 * k + L, L)], hi)

            pltpu.async_copy(xb[b], out_hbm.at[pl.ds(row0, R)], so[b])

    for c in (CHUNKS - 2, CHUNKS - 1):
        pltpu.make_async_copy(
            xb[c % NB], out_hbm.at[pl.ds(base + c * R, R)], so[c % NB]).wait()


def kernel(x, frame_indices, pe):
    xf = x.reshape(N, D)
    idx = frame_indices.reshape(N).astype(jnp.int32)
    idxp = jnp.pad(idx.reshape(N // R, R), ((0, 0), (0, 8 - R))).reshape(-1)
    pe_bf = (pe.astype(jnp.bfloat16)
             .reshape(MAX_FRAMES, D // 32, 2, 16)
             .transpose(0, 1, 3, 2)
             .reshape(MAX_FRAMES, D // 2, 2))
    pe_pk = jax.lax.bitcast_convert_type(pe_bf, jnp.int32)
    out = _pe_add(xf, idxp, pe_pk)
    return out.reshape(B, T, D)
